# baseline (speedup 1.0000x reference)
import jax
import jax.numpy as jnp
from jax.experimental import pallas as pl
from jax.experimental.pallas import tpu as pltpu

_N = 2048
_D_IN = 512
_D_H = 2048
_D_OUT = 16
_RB = 256
_NB = _N // _RB
_CB = _D_H // _NB


def _gcn_kernel(x_ref, sup_ref, w1_ref, w2_ref, wc_ref, bc_ref,
                out_ref, h1_ref, b_ref, acc_ref):
    p = pl.program_id(0)
    i = pl.program_id(1)

    @pl.when(p == 0)
    def _phase_h1():
        a = jnp.dot(sup_ref[...], x_ref[...],
                    preferred_element_type=jnp.float32)
        h1 = jnp.maximum(
            jnp.dot(a, w1_ref[...], preferred_element_type=jnp.float32), 0.0)
        h1_ref[pl.ds(i * _RB, _RB), :] = h1

    @pl.when(p == 1)
    def _phase_b():
        b_ref[:, pl.ds(i * _CB, _CB)] = jnp.dot(
            h1_ref[...], w2_ref[...], preferred_element_type=jnp.float32)

    @pl.when(p == 2)
    def _phase_h2():
        @pl.when(i == 0)
        def _init():
            acc_ref[...] = jnp.zeros_like(acc_ref)

        h2 = jnp.maximum(
            jnp.dot(sup_ref[...], b_ref[...],
                    preferred_element_type=jnp.float32),
            0.0)
        rs = jnp.sum(h2, axis=1, keepdims=True)
        acc_ref[...] += jnp.sum(rs * wc_ref[...], axis=0, keepdims=True)

        @pl.when(i == _NB - 1)
        def _final():
            logits = acc_ref[...] * (1.0 / _D_H) + bc_ref[...]
            mx = jnp.max(logits, axis=1, keepdims=True)
            e = jnp.exp(logits - mx)
            out_ref[...] = e / jnp.sum(e, axis=1, keepdims=True)


def kernel(x, support, W1, W2, Wc, bc):
    bc2 = bc.reshape(1, _D_OUT)
    last = _NB - 1
    return pl.pallas_call(
        _gcn_kernel,
        grid=(3, _NB),
        in_specs=[
            pl.BlockSpec((_N, _D_IN), lambda p, i: (0, 0)),
            pl.BlockSpec((_RB, _N),
                         lambda p, i: (jnp.where(p == 1, last, i), 0)),
            pl.BlockSpec((_D_IN, _D_H), lambda p, i: (0, 0)),
            pl.BlockSpec((_D_H, _CB),
                         lambda p, i: (0, jnp.where(p == 1, i, 0))),
            pl.BlockSpec((_RB, _D_OUT),
                         lambda p, i: (jnp.where(p == 2, i, 0), 0)),
            pl.BlockSpec((1, _D_OUT), lambda p, i: (0, 0)),
        ],
        out_specs=pl.BlockSpec((1, _D_OUT), lambda p, i: (0, 0)),
        out_shape=jax.ShapeDtypeStruct((1, _D_OUT), jnp.float32),
        scratch_shapes=[
            pltpu.VMEM((_N, _D_H), jnp.float32),
            pltpu.VMEM((_N, _D_H), jnp.float32),
            pltpu.VMEM((1, _D_OUT), jnp.float32),
        ],
        compiler_params=pltpu.CompilerParams(
            vmem_limit_bytes=60 * 1024 * 1024),
    )(x, support, W1, W2, Wc, bc2)


# speedup vs baseline: 1.0769x; 1.0769x over previous
import jax
import jax.numpy as jnp
from jax.experimental import pallas as pl
from jax.experimental.pallas import tpu as pltpu

_N = 2048
_D_IN = 512
_D_H = 2048
_D_OUT = 16
_RB = 512
_NB = _N // _RB
_CB = _D_H // _NB


def _gcn_kernel(x_ref, sup_ref, w1_ref, w2_ref, wc_ref, bc_ref,
                out_ref, h1_ref, b_ref, acc_ref):
    p = pl.program_id(0)
    i = pl.program_id(1)

    @pl.when(p == 0)
    def _phase_h1():
        a = jnp.dot(sup_ref[...], x_ref[...],
                    preferred_element_type=jnp.float32)
        h1 = jnp.maximum(
            jnp.dot(a, w1_ref[...], preferred_element_type=jnp.float32), 0.0)
        h1_ref[pl.ds(i * _RB, _RB), :] = h1

    @pl.when(p == 1)
    def _phase_b():
        b_ref[:, pl.ds(i * _CB, _CB)] = jnp.dot(
            h1_ref[...], w2_ref[...], preferred_element_type=jnp.float32)

    @pl.when(p == 2)
    def _phase_h2():
        @pl.when(i == 0)
        def _init():
            acc_ref[...] = jnp.zeros_like(acc_ref)

        h2 = jnp.maximum(
            jnp.dot(sup_ref[...], b_ref[...],
                    preferred_element_type=jnp.float32),
            0.0)
        rs = jnp.sum(h2, axis=1, keepdims=True)
        acc_ref[...] += jnp.sum(rs * wc_ref[...], axis=0, keepdims=True)

        @pl.when(i == _NB - 1)
        def _final():
            logits = acc_ref[...] * (1.0 / _D_H) + bc_ref[...]
            mx = jnp.max(logits, axis=1, keepdims=True)
            e = jnp.exp(logits - mx)
            out_ref[...] = e / jnp.sum(e, axis=1, keepdims=True)


def kernel(x, support, W1, W2, Wc, bc):
    bc2 = bc.reshape(1, _D_OUT)
    last = _NB - 1
    return pl.pallas_call(
        _gcn_kernel,
        grid=(3, _NB),
        in_specs=[
            pl.BlockSpec((_N, _D_IN), lambda p, i: (0, 0)),
            pl.BlockSpec((_RB, _N),
                         lambda p, i: (jnp.where(p == 1, last, i), 0)),
            pl.BlockSpec((_D_IN, _D_H), lambda p, i: (0, 0)),
            pl.BlockSpec((_D_H, _CB),
                         lambda p, i: (0, jnp.where(p == 1, i, 0))),
            pl.BlockSpec((_RB, _D_OUT),
                         lambda p, i: (jnp.where(p == 2, i, 0), 0)),
            pl.BlockSpec((1, _D_OUT), lambda p, i: (0, 0)),
        ],
        out_specs=pl.BlockSpec((1, _D_OUT), lambda p, i: (0, 0)),
        out_shape=jax.ShapeDtypeStruct((1, _D_OUT), jnp.float32),
        scratch_shapes=[
            pltpu.VMEM((_N, _D_H), jnp.float32),
            pltpu.VMEM((_N, _D_H), jnp.float32),
            pltpu.VMEM((1, _D_OUT), jnp.float32),
        ],
        compiler_params=pltpu.CompilerParams(
            vmem_limit_bytes=63 * 1024 * 1024),
    )(x, support, W1, W2, Wc, bc2)
